# baseline (speedup 1.0000x reference)
import functools

import jax
import jax.numpy as jnp
from jax import lax
from jax.experimental import pallas as pl
from jax.experimental.pallas import tpu as pltpu
from jax.experimental.pallas import tpu_sc as plsc

EMBED = 128
MAX_POS = 4096
TYPE_VOCAB = 2

NC, NS, LANES = 2, 16, 16
NW = NC * NS
CH = 128
ROW_UNROLL = 4
TBLK = 1024


def _combined_body(pos_ref, type_ref, out_ref):
    i = pl.program_id(0)
    t = i // (MAX_POS // TBLK)
    rows = type_ref[...]
    row = jnp.where(t == 0, rows[0:1, :], rows[1:2, :])
    out_ref[...] = pos_ref[...] + row


def _build_combined(pos_emb, type_emb):
    k = MAX_POS // TBLK
    return pl.pallas_call(
        _combined_body,
        grid=(TYPE_VOCAB * k,),
        in_specs=[
            pl.BlockSpec((TBLK, EMBED), lambda i: (i % k, 0)),
            pl.BlockSpec((TYPE_VOCAB, EMBED), lambda i: (0, 0)),
        ],
        out_specs=pl.BlockSpec((TBLK, EMBED), lambda i: (i, 0)),
        out_shape=jax.ShapeDtypeStruct((TYPE_VOCAB * MAX_POS, EMBED), jnp.float32),
    )(pos_emb, type_emb)


def _gather_sum(word_emb, comb_table, word_ids, pos_ids, type_ids):
    B, S = word_ids.shape
    n = B * S
    b_per_w = n // NW
    n_chunks = b_per_w // CH
    w_per_row = S // b_per_w
    mesh = plsc.VectorSubcoreMesh(core_axis_name="c", subcore_axis_name="s")

    @functools.partial(
        pl.kernel,
        mesh=mesh,
        out_type=jax.ShapeDtypeStruct((n, EMBED), jnp.float32),
        scratch_types=[
            pltpu.VMEM((CH,), jnp.int32),
            pltpu.VMEM((CH,), jnp.int32),
            pltpu.VMEM((CH,), jnp.int32),
            pltpu.VMEM((CH,), jnp.int32),
            pltpu.VMEM((CH,), jnp.int32),
            pltpu.VMEM((CH,), jnp.int32),
            pltpu.VMEM((CH, EMBED), jnp.float32),
            pltpu.VMEM((CH, EMBED), jnp.float32),
            pltpu.VMEM((CH, EMBED), jnp.float32),
            pltpu.VMEM((CH, EMBED), jnp.float32),
            pltpu.SemaphoreType.DMA,
            pltpu.SemaphoreType.DMA,
            pltpu.SemaphoreType.DMA,
            pltpu.SemaphoreType.DMA,
            pltpu.SemaphoreType.DMA,
            pltpu.SemaphoreType.DMA,
            pltpu.SemaphoreType.DMA,
            pltpu.SemaphoreType.DMA,
        ],
    )
    def k(word_hbm, comb_hbm, wid_hbm, pid_hbm, tid_hbm, out_hbm,
          wi0, wi1, pi0, pi1, ti0, ti1, a0, a1, b0, b1,
          si0, si1, ga0, ga1, gb0, gb1, so0, so1):
        wid = lax.axis_index("c") * NS + lax.axis_index("s")
        base = wid * b_per_w
        row = wid // w_per_row
        col0 = (wid % w_per_row) * b_per_w
        wi = (wi0, wi1)
        pi = (pi0, pi1)
        ti = (ti0, ti1)
        a = (a0, a1)
        b = (b0, b1)
        si = (si0, si1)
        ga = (ga0, ga1)
        gb = (gb0, gb1)
        so = (so0, so1)

        def start_ids(c):
            p = c % 2
            cols = pl.ds(col0 + c * CH, CH)
            return (
                pltpu.async_copy(wid_hbm.at[row, cols], wi[p], si[p]),
                pltpu.async_copy(pid_hbm.at[row, cols], pi[p], si[p]),
                pltpu.async_copy(tid_hbm.at[row, cols], ti[p], si[p]),
            )

        def combine_ids(c):
            p = c % 2
            pv, tv = pi[p], ti[p]
            for j in range(CH // LANES):
                s = pl.ds(j * LANES, LANES)
                pv[s] = tv[s] * MAX_POS + pv[s]

        def start_gathers(c):
            p = c % 2
            return (
                pltpu.async_copy(word_hbm.at[wi[p]], a[p], ga[p]),
                pltpu.async_copy(comb_hbm.at[pi[p]], b[p], gb[p]),
            )

        ids_pend = {0: start_ids(0)}
        for h in ids_pend.pop(0):
            h.wait()
        combine_ids(0)
        gat_pend = {0: start_gathers(0)}
        ids_pend[1] = start_ids(1)
        out_pend = {}

        for c in range(n_chunks):
            p = c % 2
            if c + 1 < n_chunks:
                for h in ids_pend.pop(c + 1):
                    h.wait()
                combine_ids(c + 1)
                if c - 1 >= 0:
                    out_pend.pop(c - 1).wait()
                gat_pend[c + 1] = start_gathers(c + 1)
            cpa, cpb = gat_pend.pop(c)
            cpa.wait()
            cpb.wait()
            if c + 2 < n_chunks:
                ids_pend[c + 2] = start_ids(c + 2)

            av, bv = a[p], b[p]

            @pl.loop(0, CH, step=ROW_UNROLL)
            def _(r):
                for rr in range(ROW_UNROLL):
                    for j in range(EMBED // LANES):
                        s = pl.ds(j * LANES, LANES)
                        plsc.addupdate(av.at[r + rr, s], bv[r + rr, s])

            out_pend[c] = pltpu.async_copy(
                av, out_hbm.at[pl.ds(base + c * CH, CH)], so[p])
        for c in sorted(out_pend):
            out_pend.pop(c).wait()

    return k(word_emb, comb_table, word_ids, pos_ids, type_ids)


def kernel(input_ids, token_type_ids, position_ids, word_emb, pos_emb, type_emb):
    B, S = input_ids.shape
    comb_table = _build_combined(pos_emb, type_emb)
    out = _gather_sum(word_emb, comb_table,
                      input_ids.astype(jnp.int32),
                      position_ids.astype(jnp.int32),
                      token_type_ids.astype(jnp.int32))
    return out.reshape(B, S, EMBED)


# speedup vs baseline: 1.0840x; 1.0840x over previous
import functools

import jax
import jax.numpy as jnp
from jax import lax
from jax.experimental import pallas as pl
from jax.experimental.pallas import tpu as pltpu
from jax.experimental.pallas import tpu_sc as plsc

EMBED = 128
MAX_POS = 4096
TYPE_VOCAB = 2

NC, NS, LANES = 2, 16, 16
NW = NC * NS
CH = 128
ROW_UNROLL = 4
TBLK = 4096


def _combined_body(pos_ref, type_ref, out_ref):
    i = pl.program_id(0)
    t = i // (MAX_POS // TBLK)
    rows = type_ref[...]
    row = jnp.where(t == 0, rows[0:1, :], rows[1:2, :])
    out_ref[...] = pos_ref[...] + row


def _build_combined(pos_emb, type_emb):
    k = MAX_POS // TBLK
    return pl.pallas_call(
        _combined_body,
        grid=(TYPE_VOCAB * k,),
        in_specs=[
            pl.BlockSpec((TBLK, EMBED), lambda i: (i % k, 0)),
            pl.BlockSpec((TYPE_VOCAB, EMBED), lambda i: (0, 0)),
        ],
        out_specs=pl.BlockSpec((TBLK, EMBED), lambda i: (i, 0)),
        out_shape=jax.ShapeDtypeStruct((TYPE_VOCAB * MAX_POS, EMBED), jnp.float32),
    )(pos_emb, type_emb)


def _gather_sum(word_emb, comb_table, word_ids, pos_ids, type_ids):
    B, S = word_ids.shape
    n = B * S
    b_per_w = n // NW
    n_chunks = b_per_w // CH
    w_per_row = S // b_per_w
    mesh = plsc.VectorSubcoreMesh(core_axis_name="c", subcore_axis_name="s")

    @functools.partial(
        pl.kernel,
        mesh=mesh,
        out_type=jax.ShapeDtypeStruct((n, EMBED), jnp.float32),
        scratch_types=[
            pltpu.VMEM((CH,), jnp.int32),
            pltpu.VMEM((CH,), jnp.int32),
            pltpu.VMEM((CH,), jnp.int32),
            pltpu.VMEM((CH,), jnp.int32),
            pltpu.VMEM((CH,), jnp.int32),
            pltpu.VMEM((CH,), jnp.int32),
            pltpu.VMEM((CH, EMBED), jnp.float32),
            pltpu.VMEM((CH, EMBED), jnp.float32),
            pltpu.VMEM((CH, EMBED), jnp.float32),
            pltpu.VMEM((CH, EMBED), jnp.float32),
            pltpu.SemaphoreType.DMA,
            pltpu.SemaphoreType.DMA,
            pltpu.SemaphoreType.DMA,
            pltpu.SemaphoreType.DMA,
            pltpu.SemaphoreType.DMA,
            pltpu.SemaphoreType.DMA,
            pltpu.SemaphoreType.DMA,
            pltpu.SemaphoreType.DMA,
        ],
    )
    def k(word_hbm, comb_hbm, wid_hbm, pid_hbm, tid_hbm, out_hbm,
          wi0, wi1, pi0, pi1, ti0, ti1, a0, a1, b0, b1,
          si0, si1, ga0, ga1, gb0, gb1, so0, so1):
        wid = lax.axis_index("c") * NS + lax.axis_index("s")
        base = wid * b_per_w
        row = wid // w_per_row
        col0 = (wid % w_per_row) * b_per_w
        wi = (wi0, wi1)
        pi = (pi0, pi1)
        ti = (ti0, ti1)
        a = (a0, a1)
        b = (b0, b1)
        si = (si0, si1)
        ga = (ga0, ga1)
        gb = (gb0, gb1)
        so = (so0, so1)

        def start_ids(c):
            p = c % 2
            cols = pl.ds(col0 + c * CH, CH)
            return (
                pltpu.async_copy(wid_hbm.at[row, cols], wi[p], si[p]),
                pltpu.async_copy(pid_hbm.at[row, cols], pi[p], si[p]),
                pltpu.async_copy(tid_hbm.at[row, cols], ti[p], si[p]),
            )

        def combine_ids(c):
            p = c % 2
            pv, tv = pi[p], ti[p]
            for j in range(CH // LANES):
                s = pl.ds(j * LANES, LANES)
                pv[s] = tv[s] * MAX_POS + pv[s]

        def start_gathers(c):
            p = c % 2
            return (
                pltpu.async_copy(word_hbm.at[wi[p]], a[p], ga[p]),
                pltpu.async_copy(comb_hbm.at[pi[p]], b[p], gb[p]),
            )

        ids_pend = {0: start_ids(0)}
        for h in ids_pend.pop(0):
            h.wait()
        combine_ids(0)
        gat_pend = {0: start_gathers(0)}
        ids_pend[1] = start_ids(1)
        out_pend = {}

        for c in range(n_chunks):
            p = c % 2
            if c + 1 < n_chunks:
                for h in ids_pend.pop(c + 1):
                    h.wait()
                combine_ids(c + 1)
                if c - 1 >= 0:
                    out_pend.pop(c - 1).wait()
                gat_pend[c + 1] = start_gathers(c + 1)
            cpa, cpb = gat_pend.pop(c)
            cpa.wait()
            cpb.wait()
            if c + 2 < n_chunks:
                ids_pend[c + 2] = start_ids(c + 2)

            av, bv = a[p], b[p]

            @pl.loop(0, CH, step=ROW_UNROLL)
            def _(r):
                for rr in range(ROW_UNROLL):
                    for j in range(EMBED // LANES):
                        s = pl.ds(j * LANES, LANES)
                        plsc.addupdate(av.at[r + rr, s], bv[r + rr, s])

            out_pend[c] = pltpu.async_copy(
                av, out_hbm.at[pl.ds(base + c * CH, CH)], so[p])
        for c in sorted(out_pend):
            out_pend.pop(c).wait()

    return k(word_emb, comb_table, word_ids, pos_ids, type_ids)


def kernel(input_ids, token_type_ids, position_ids, word_emb, pos_emb, type_emb):
    B, S = input_ids.shape
    comb_table = _build_combined(pos_emb, type_emb)
    out = _gather_sum(word_emb, comb_table,
                      input_ids.astype(jnp.int32),
                      position_ids.astype(jnp.int32),
                      token_type_ids.astype(jnp.int32))
    return out.reshape(B, S, EMBED)
